# baseline (speedup 1.0000x reference)
import functools

import jax
import jax.numpy as jnp
from jax import lax
from jax.experimental import pallas as pl
from jax.experimental.pallas import tpu as pltpu
from jax.experimental.pallas import tpu_sc as plsc


def _contract_body(basis_ref, cmat_ref, w_ref):
    w_ref[...] = jnp.dot(
        basis_ref[...], cmat_ref[...], preferred_element_type=jnp.float32
    )


def _materialize_weight(basis2, cmat, rows_per_block):
    n, de = basis2.shape
    d = cmat.shape[1]
    return pl.pallas_call(
        _contract_body,
        grid=(n // rows_per_block,),
        in_specs=[
            pl.BlockSpec((rows_per_block, de), lambda i: (i, 0)),
            pl.BlockSpec((de, d), lambda i: (0, 0)),
        ],
        out_specs=pl.BlockSpec((rows_per_block, d), lambda i: (i, 0)),
        out_shape=jax.ShapeDtypeStruct((n, d), jnp.float32),
    )(basis2, cmat)


def _sc_gather(weight, idx_t, batch, hist):
    info = plsc.get_sparse_core_info()
    nc, ns = info.num_cores, info.num_subcores
    nw = nc * ns
    d = weight.shape[1]
    b_per_w = batch // nw

    mesh = plsc.VectorSubcoreMesh(core_axis_name="c", subcore_axis_name="s")

    @functools.partial(
        pl.kernel,
        mesh=mesh,
        out_type=jax.ShapeDtypeStruct((batch, hist, d), jnp.float32),
        scratch_types=[
            pltpu.VMEM((b_per_w,), jnp.int32),
            pltpu.VMEM((b_per_w, d), jnp.float32),
            pltpu.SemaphoreType.DMA,
        ],
        compiler_params=pltpu.CompilerParams(use_tc_tiling_on_sc=False),
    )
    def k(idx_hbm, table_hbm, out_hbm, idx_v, rows_v, sem):
        wid = lax.axis_index("s") * nc + lax.axis_index("c")
        base = wid * b_per_w

        def step(l, _):
            pltpu.sync_copy(idx_hbm.at[pl.ds(l * batch + base, b_per_w)], idx_v)
            pltpu.async_copy(table_hbm.at[idx_v], rows_v, sem).wait()
            pltpu.sync_copy(rows_v, out_hbm.at[pl.ds(base, b_per_w), l])
            return _

        lax.fori_loop(0, hist, step, None)

    return k(idx_t, weight)


def kernel(indices, coef, basis):
    n, d, e = basis.shape
    basis2 = basis.reshape(n, d * e)
    cmat = (jnp.eye(d, dtype=coef.dtype)[:, None, :] * coef[None, :, None]).reshape(
        d * e, d
    )
    weight = _materialize_weight(basis2, cmat, rows_per_block=5000)
    idx_t = indices.astype(jnp.int32).T.reshape(-1)
    return _sc_gather(weight, idx_t, indices.shape[0], indices.shape[1])
